# baseline (speedup 1.0000x reference)
import jax
import jax.numpy as jnp
from jax import lax
from jax.experimental import pallas as pl
from jax.experimental.pallas import tpu as pltpu
from jax.experimental.pallas import tpu_sc as plsc

N = 10000
E = 320000
D = 128
DH = 64
BETA_ = 0.5
C_ = 1.0

NC = 2
NS = 16
SUB = 128
CR = E // SUB
SLAB = 160
SLAB_LD = 168
CR_PAD = 15 * SLAB + SLAB_LD
N_PAD = 10240
ROWS_PER_TILE = N_PAD // NS
QUADS = SLAB // 4


def _mm_body(x_ref, w_ref, o_ref):
    r = jnp.dot(x_ref[...], w_ref[...], preferred_element_type=jnp.float32)
    o_ref[0] = r[:, :DH]
    o_ref[1] = r[:, DH:]


def _matmul(x, W):
    return pl.pallas_call(
        _mm_body,
        grid=(10,),
        in_specs=[
            pl.BlockSpec((N // 10, D), lambda i: (i, 0)),
            pl.BlockSpec((D, D), lambda i: (0, 0)),
        ],
        out_specs=pl.BlockSpec((2, N // 10, DH), lambda i: (0, i, 0)),
        out_shape=jax.ShapeDtypeStruct((2, N, DH), jnp.float32),
    )(x, W)


def _mix_body(p_ref, b_ref, o_ref):
    z = jnp.concatenate([p_ref[0], p_ref[1]], axis=-1) + b_ref[...]
    o_ref[...] = BETA_ * z + (C_ - BETA_) * jnp.maximum(z, 0.0)


def _mix(partials, b):
    return pl.pallas_call(
        _mix_body,
        grid=(10,),
        in_specs=[
            pl.BlockSpec((2, N // 10, DH), lambda i: (0, i, 0)),
            pl.BlockSpec((1, D), lambda i: (0, 0)),
        ],
        out_specs=pl.BlockSpec((N // 10, D), lambda i: (i, 0)),
        out_shape=jax.ShapeDtypeStruct((N, D), jnp.float32),
    )(partials, b.reshape(1, D))


def _sc_body(xlin, packed, ews, out, acc,
             packed_v, ew_v,
             rows_a, rows_b, rows_c, rows_d,
             sra, dra, srb, drb, src, drc, srd, drd,
             sem_a, sem_b, sem_c, sem_d):
    c = lax.axis_index("c")
    s = lax.axis_index("s")
    coff = c * N

    def zero_rows(buf):
        def zrow(i, carry):
            for cb in range(DH // 16):
                buf[i, pl.ds(cb * 16, 16)] = jnp.zeros((16,), jnp.float32)
            return carry
        lax.fori_loop(0, SUB, zrow, 0)

    zero_rows(rows_a)
    base_n = s * ROWS_PER_TILE
    for k in range(ROWS_PER_TILE // SUB):
        pltpu.sync_copy(rows_a, acc.at[pl.ds(base_n + k * SUB, SUB)])
    zero_rows(rows_c)
    zero_rows(rows_d)
    for v in range(SUB // 16):
        drc[pl.ds(v * 16, 16)] = jnp.zeros((16,), jnp.int32)
        drd[pl.ds(v * 16, 16)] = jnp.zeros((16,), jnp.int32)
    plsc.subcore_barrier()

    start = s * SLAB
    pltpu.sync_copy(packed.at[pl.ds(start, SLAB_LD)], packed_v)
    pltpu.sync_copy(ews.at[pl.ds(start, SLAB_LD)], ew_v)

    def unpack(j, src_r, dst_r):
        cv = jnp.broadcast_to(coff, (16,)).astype(jnp.int32)
        for v in range(SUB // 16):
            p = packed_v[j, pl.ds(v * 16, 16)]
            src_r[pl.ds(v * 16, 16)] = (p & 0x3FFF) + cv
            dst_r[pl.ds(v * 16, 16)] = lax.shift_right_logical(p, 14)

    def gather(buf, src_r, sem):
        pltpu.async_copy(xlin.at[src_r], buf, sem)

    def wait_g(buf, src_r, sem):
        pltpu.make_async_copy(xlin.at[src_r], buf, sem).wait()

    def scatter(buf, dst_r, sem):
        pltpu.async_copy(buf, acc.at[dst_r], sem, add=True)

    def wait_s(buf, dst_r, sem):
        pltpu.make_async_copy(buf, acc.at[dst_r], sem).wait()

    def scale(j, buf):
        def scale16(q, carry2):
            ewv = ew_v[j, pl.ds(q * 16, 16)]
            for e in range(16):
                wv = jnp.broadcast_to(ewv[e], (16,))
                for cb in range(DH // 16):
                    r = buf[q * 16 + e, pl.ds(cb * 16, 16)]
                    buf[q * 16 + e, pl.ds(cb * 16, 16)] = r * wv
            return carry2
        lax.fori_loop(0, SUB // 16, scale16, 0)

    scatter(rows_c, drc, sem_c)
    scatter(rows_d, drd, sem_d)
    unpack(0, sra, dra)
    gather(rows_a, sra, sem_a)
    unpack(1, srb, drb)
    gather(rows_b, srb, sem_b)

    def quad(q, carry):
        j = 4 * q
        wait_s(rows_c, drc, sem_c)
        unpack(j + 2, src, drc)
        gather(rows_c, src, sem_c)
        wait_s(rows_d, drd, sem_d)
        unpack(j + 3, srd, drd)
        gather(rows_d, srd, sem_d)
        wait_g(rows_a, sra, sem_a)
        scale(j, rows_a)
        scatter(rows_a, dra, sem_a)
        wait_g(rows_b, srb, sem_b)
        scale(j + 1, rows_b)
        scatter(rows_b, drb, sem_b)
        wait_s(rows_a, dra, sem_a)
        unpack(j + 4, sra, dra)
        gather(rows_a, sra, sem_a)
        wait_s(rows_b, drb, sem_b)
        unpack(j + 5, srb, drb)
        gather(rows_b, srb, sem_b)
        wait_g(rows_c, src, sem_c)
        scale(j + 2, rows_c)
        scatter(rows_c, drc, sem_c)
        wait_g(rows_d, srd, sem_d)
        scale(j + 3, rows_d)
        scatter(rows_d, drd, sem_d)
        return carry

    lax.fori_loop(0, QUADS, quad, 0)
    wait_g(rows_a, sra, sem_a)
    wait_g(rows_b, srb, sem_b)
    wait_s(rows_c, drc, sem_c)
    wait_s(rows_d, drd, sem_d)
    plsc.subcore_barrier()

    pltpu.sync_copy(acc.at[pl.ds(base_n, ROWS_PER_TILE)],
                    out.at[c, pl.ds(base_n, ROWS_PER_TILE)])


def _scatter_gather(xlin_flat, packed, ews):
    mesh = plsc.VectorSubcoreMesh(core_axis_name="c", subcore_axis_name="s")
    return pl.kernel(
        _sc_body,
        out_type=jax.ShapeDtypeStruct((NC, N_PAD, DH), jnp.float32),
        mesh=mesh,
        compiler_params=pltpu.CompilerParams(use_tc_tiling_on_sc=False),
        scratch_types=[
            pltpu.VMEM_SHARED((N_PAD, DH), jnp.float32),
            pltpu.VMEM((SLAB_LD, SUB), jnp.int32),
            pltpu.VMEM((SLAB_LD, SUB), jnp.float32),
            pltpu.VMEM((SUB, DH), jnp.float32),
            pltpu.VMEM((SUB, DH), jnp.float32),
            pltpu.VMEM((SUB, DH), jnp.float32),
            pltpu.VMEM((SUB, DH), jnp.float32),
            pltpu.VMEM((SUB,), jnp.int32),
            pltpu.VMEM((SUB,), jnp.int32),
            pltpu.VMEM((SUB,), jnp.int32),
            pltpu.VMEM((SUB,), jnp.int32),
            pltpu.VMEM((SUB,), jnp.int32),
            pltpu.VMEM((SUB,), jnp.int32),
            pltpu.VMEM((SUB,), jnp.int32),
            pltpu.VMEM((SUB,), jnp.int32),
            pltpu.SemaphoreType.DMA,
            pltpu.SemaphoreType.DMA,
            pltpu.SemaphoreType.DMA,
            pltpu.SemaphoreType.DMA,
        ],
    )(xlin_flat, packed, ews)


def kernel(x, edge_index, edge_weight, W, b):
    src = edge_index[0].astype(jnp.int32)
    dst = edge_index[1].astype(jnp.int32)
    packed = (src | (dst << 14))
    pad = CR_PAD * SUB - E
    packed = jnp.pad(packed, (0, pad)).reshape(CR_PAD, SUB)
    ew = jnp.pad(edge_weight, (0, pad)).reshape(CR_PAD, SUB)
    xlin = _matmul(x, W).reshape(NC * N, DH)
    partials = _scatter_gather(xlin, packed, ew)
    return _mix(partials, b)
